# baseline (speedup 1.0000x reference)
import functools

import jax
import jax.numpy as jnp
from jax.experimental import pallas as pl
from jax.experimental.pallas import tpu as pltpu


def _crelu(x):
    return jnp.clip(x, 0.0, 1.0)


def _nnue_body(wf_ref, bf_ref, w8_ref, stm_ref, ftb8_ref, l1wT_ref, l1b_ref,
               l2wT_ref, l2b_ref, l3wT_ref, l3b_ref, out_ref,
               accA_ref, accC_ref):
    i = pl.program_id(0)

    @pl.when(i == 0)
    def _init():
        accA_ref[...] = jnp.zeros_like(accA_ref)
        accC_ref[...] = jnp.zeros_like(accC_ref)

    w8 = w8_ref[...]
    accA_ref[...] += jnp.dot(wf_ref[...], w8,
                             preferred_element_type=jnp.float32)
    accC_ref[...] += jnp.dot(bf_ref[...], w8,
                             preferred_element_type=jnp.float32)

    @pl.when(i == pl.num_programs(0) - 1)
    def _tail():
        A = accA_ref[...]
        C = accC_ref[...]
        lane = jax.lax.broadcasted_iota(jnp.int32, A.shape, 1)
        first_half = lane < 4
        wb = jnp.where(first_half, A, C)
        bw = jnp.where(first_half, C, A)
        stm = stm_ref[...]
        acc = stm * wb + (1.0 - stm) * bw + ftb8_ref[...]
        x = _crelu(acc)
        x = _crelu(jnp.dot(x, l1wT_ref[...],
                           preferred_element_type=jnp.float32) + l1b_ref[...])
        x = _crelu(jnp.dot(x, l2wT_ref[...],
                           preferred_element_type=jnp.float32) + l2b_ref[...])
        out_ref[...] = jnp.dot(x, l3wT_ref[...],
                               preferred_element_type=jnp.float32) + l3b_ref[...]


@functools.partial(jax.jit, static_argnames=("fc",))
def _nnue(wfts, bfts, stm, ft_w, ft_b, l1_w, l1_b, l2_w, l2_b, l3_w, l3_b,
          fc=2048):
    B, F = wfts.shape
    ftwT = ft_w.T
    w8 = jnp.concatenate([ftwT, ftwT], axis=1)
    ftb8 = jnp.concatenate([ft_b, ft_b]).reshape(1, 8)
    grid = (F // fc,)
    return pl.pallas_call(
        _nnue_body,
        grid=grid,
        in_specs=[
            pl.BlockSpec((B, fc), lambda i: (0, i)),
            pl.BlockSpec((B, fc), lambda i: (0, i)),
            pl.BlockSpec((fc, 8), lambda i: (i, 0)),
            pl.BlockSpec((B, 1), lambda i: (0, 0)),
            pl.BlockSpec((1, 8), lambda i: (0, 0)),
            pl.BlockSpec((8, 8), lambda i: (0, 0)),
            pl.BlockSpec((1, 8), lambda i: (0, 0)),
            pl.BlockSpec((8, 8), lambda i: (0, 0)),
            pl.BlockSpec((1, 8), lambda i: (0, 0)),
            pl.BlockSpec((8, 1), lambda i: (0, 0)),
            pl.BlockSpec((1, 1), lambda i: (0, 0)),
        ],
        out_specs=pl.BlockSpec((B, 1), lambda i: (0, 0)),
        out_shape=jax.ShapeDtypeStruct((B, 1), jnp.float32),
        scratch_shapes=[
            pltpu.VMEM((B, 8), jnp.float32),
            pltpu.VMEM((B, 8), jnp.float32),
        ],
        compiler_params=pltpu.CompilerParams(
            dimension_semantics=("arbitrary",),
        ),
    )(wfts, bfts, w8, stm, ftb8,
      l1_w.T, l1_b.reshape(1, 8),
      l2_w.T, l2_b.reshape(1, 8),
      l3_w.T, l3_b.reshape(1, 1))


def kernel(wfts, bfts, stm, ft_w, ft_b, l1_w, l1_b, l2_w, l2_b, l3_w, l3_b):
    return _nnue(wfts, bfts, stm, ft_w, ft_b,
                 l1_w, l1_b, l2_w, l2_b, l3_w, l3_b)
